# baseline (speedup 1.0000x reference)
import functools

import jax
import jax.numpy as jnp
from jax import lax
from jax.experimental import pallas as pl
from jax.experimental.pallas import tpu as pltpu
from jax.experimental.pallas import tpu_sc as plsc

NC, NS = 2, 16
NW = NC * NS
CH = 40
NB = 50
RB = 5
DW = 16


def _sc_aggregate(n_nodes, d_in, n_edges):
    et = n_edges // NW
    nch = et // CH
    nblk = nch // NB
    blk_e = NB * CH
    nrnd = NB // RB
    rows_per_tile = n_nodes // NS
    zrows = 25
    drows = 125
    assert et % CH == 0 and nch % NB == 0 and NB % RB == 0
    assert rows_per_tile % zrows == 0 and rows_per_tile % drows == 0

    mesh = plsc.VectorSubcoreMesh(
        core_axis_name="c", subcore_axis_name="s", num_cores=NC,
        num_subcores=NS)

    @functools.partial(
        pl.kernel,
        out_type=(jax.ShapeDtypeStruct((NC, n_nodes, d_in), jnp.float32),
                  jax.ShapeDtypeStruct((NC, n_nodes, DW), jnp.float32)),
        mesh=mesh,
        scratch_types=[
            pltpu.VMEM((blk_e,), jnp.int32),
            pltpu.VMEM((blk_e,), jnp.int32),
            [pltpu.VMEM((CH, d_in), jnp.float32) for _ in range(RB)],
            pltpu.VMEM((CH, DW), jnp.float32),
            pltpu.VMEM((zrows, d_in), jnp.float32),
            pltpu.VMEM((zrows, DW), jnp.float32),
            pltpu.VMEM_SHARED((n_nodes, d_in), jnp.float32),
            pltpu.VMEM_SHARED((n_nodes, DW), jnp.float32),
            [pltpu.SemaphoreType.DMA for _ in range(RB)],
            [pltpu.SemaphoreType.DMA for _ in range(RB)],
            pltpu.SemaphoreType.DMA,
            pltpu.SemaphoreType.DMA,
        ],
        compiler_params=pltpu.CompilerParams(use_tc_tiling_on_sc=False),
    )
    def agg_kernel(x_hbm, edge_hbm, out_hbm, deg_hbm,
                   sidx, didx, rows, ones, zbuf, zbufd, acc, deg,
                   gsem, ssem, dsem, hsem):
        c = lax.axis_index("c")
        s = lax.axis_index("s")
        wid = c * NS + s
        ebase = wid * et
        nbase = s * rows_per_tile

        def sid(j):
            return sidx.at[pl.ds(j * CH, CH)]

        def did(j):
            return didx.at[pl.ds(j * CH, CH)]

        @pl.loop(0, zrows)
        def _zero_rows(r):
            for j in range(d_in // 16):
                zbuf[r, pl.ds(j * 16, 16)] = jnp.zeros((16,), jnp.float32)
            zbufd[r, :] = jnp.zeros((16,), jnp.float32)

        @pl.loop(0, CH)
        def _one_rows(r):
            ones[r, :] = jnp.ones((16,), jnp.float32)

        @pl.loop(0, rows_per_tile // zrows)
        def _zero_acc(i):
            pltpu.async_copy(zbuf, acc.at[pl.ds(nbase + i * zrows, zrows)],
                             hsem)
            pltpu.async_copy(zbufd, deg.at[pl.ds(nbase + i * zrows, zrows)],
                             hsem)

        @pl.loop(0, rows_per_tile // zrows)
        def _zero_drain(i):
            pltpu.make_async_copy(zbuf, acc.at[pl.ds(nbase, zrows)],
                                  hsem).wait()
            pltpu.make_async_copy(zbufd, deg.at[pl.ds(nbase, zrows)],
                                  hsem).wait()

        plsc.subcore_barrier()

        @pl.loop(0, nblk)
        def _block(b):
            e0 = ebase + b * blk_e
            pltpu.async_copy(edge_hbm.at[0, pl.ds(e0, blk_e)], sidx, hsem)
            pltpu.async_copy(edge_hbm.at[1, pl.ds(e0, blk_e)], didx, hsem)
            pltpu.make_async_copy(edge_hbm.at[0, pl.ds(e0, blk_e)], sidx,
                                  hsem).wait()
            pltpu.make_async_copy(edge_hbm.at[1, pl.ds(e0, blk_e)], didx,
                                  hsem).wait()

            for k in range(RB - 1):
                pltpu.async_copy(x_hbm.at[sid(k)], rows[k], gsem[k])

            @pl.loop(0, nrnd)
            def _round(t):
                j0 = RB * t
                for k in range(RB):
                    j = j0 + k
                    pltpu.make_async_copy(x_hbm.at[sid(j)], rows[k],
                                          gsem[k]).wait()
                    pltpu.async_copy(rows[k], acc.at[did(j)], ssem[k],
                                     add=True)
                    pltpu.async_copy(ones, deg.at[did(j)], dsem, add=True)
                    kp = (k + RB - 1) % RB
                    if k == 0:
                        @pl.when(t > 0)
                        def _w():
                            pltpu.make_async_copy(rows[kp], acc.at[did(0)],
                                                  ssem[kp]).wait()
                    else:
                        pltpu.make_async_copy(rows[kp], acc.at[did(0)],
                                              ssem[kp]).wait()
                    pltpu.make_async_copy(ones, deg.at[did(0)], dsem).wait()
                    if k == 0:
                        pltpu.async_copy(x_hbm.at[sid(j + RB - 1)], rows[kp],
                                         gsem[kp])
                    else:
                        @pl.when(t < nrnd - 1)
                        def _g():
                            pltpu.async_copy(x_hbm.at[sid(j + RB - 1)],
                                             rows[kp], gsem[kp])

            pltpu.make_async_copy(rows[RB - 1], acc.at[did(0)],
                                  ssem[RB - 1]).wait()

        plsc.subcore_barrier()

        @pl.loop(0, rows_per_tile // drows)
        def _dump(i):
            r0 = nbase + i * drows
            pltpu.async_copy(acc.at[pl.ds(r0, drows)],
                             out_hbm.at[c, pl.ds(r0, drows)], hsem)

        pltpu.async_copy(deg.at[pl.ds(nbase, rows_per_tile)],
                         deg_hbm.at[c, pl.ds(nbase, rows_per_tile)], hsem)

        @pl.loop(0, rows_per_tile // drows)
        def _dump_drain(i):
            pltpu.make_async_copy(acc.at[pl.ds(nbase, drows)],
                                  out_hbm.at[c, pl.ds(nbase, drows)],
                                  hsem).wait()

        pltpu.make_async_copy(deg.at[pl.ds(nbase, rows_per_tile)],
                              deg_hbm.at[c, pl.ds(nbase, rows_per_tile)],
                              hsem).wait()

    return agg_kernel


def _tc_combine(n_nodes, d_in, d_out, blk):
    grid = (n_nodes // blk,)

    def body(x_ref, pagg_ref, pdeg_ref, wr_ref, wn_ref, b_ref, o_ref):
        ps = pagg_ref[0] + pagg_ref[1]
        deg = pdeg_ref[0, :, :1] + pdeg_ref[1, :, :1]
        agg = ps / jnp.maximum(deg, 1.0)
        o_ref[...] = (
            jnp.dot(x_ref[...], wr_ref[...], preferred_element_type=jnp.float32)
            + jnp.dot(agg, wn_ref[...], preferred_element_type=jnp.float32)
            + b_ref[...])

    return pl.pallas_call(
        body,
        grid=grid,
        in_specs=[
            pl.BlockSpec((blk, d_in), lambda i: (i, 0)),
            pl.BlockSpec((NC, blk, d_in), lambda i: (0, i, 0)),
            pl.BlockSpec((NC, blk, DW), lambda i: (0, i, 0)),
            pl.BlockSpec((d_in, d_out), lambda i: (0, 0)),
            pl.BlockSpec((d_in, d_out), lambda i: (0, 0)),
            pl.BlockSpec((1, d_out), lambda i: (0, 0)),
        ],
        out_specs=pl.BlockSpec((blk, d_out), lambda i: (i, 0)),
        out_shape=jax.ShapeDtypeStruct((n_nodes, d_out), jnp.float32),
    )


def kernel(x, edge_index, W_root, W_neigh, b):
    n, d_in = x.shape
    e = edge_index.shape[1]
    d_out = W_root.shape[1]

    pagg, pdeg = _sc_aggregate(n, d_in, e)(x, edge_index)
    return _tc_combine(n, d_in, d_out, 1000)(
        x, pagg, pdeg, W_root, W_neigh, b.reshape(1, d_out))


# speedup vs baseline: 1.0030x; 1.0030x over previous
import functools

import jax
import jax.numpy as jnp
from jax import lax
from jax.experimental import pallas as pl
from jax.experimental.pallas import tpu as pltpu
from jax.experimental.pallas import tpu_sc as plsc

NC, NS = 2, 16
NW = NC * NS
CH = 40
NB = 50
RB = 5
DW = 16


def _sc_aggregate(n_nodes, d_in, n_edges):
    et = n_edges // NW
    nch = et // CH
    nblk = nch // NB
    blk_e = NB * CH
    nrnd = NB // RB
    rows_per_tile = n_nodes // NS
    zrows = 25
    drows = 125
    assert et % CH == 0 and nch % NB == 0 and NB % RB == 0
    assert rows_per_tile % zrows == 0 and rows_per_tile % drows == 0

    mesh = plsc.VectorSubcoreMesh(
        core_axis_name="c", subcore_axis_name="s", num_cores=NC,
        num_subcores=NS)

    @functools.partial(
        pl.kernel,
        out_type=(jax.ShapeDtypeStruct((NC, n_nodes, d_in), jnp.float32),
                  jax.ShapeDtypeStruct((NC, n_nodes, DW), jnp.float32)),
        mesh=mesh,
        scratch_types=[
            pltpu.VMEM((blk_e,), jnp.int32),
            pltpu.VMEM((blk_e,), jnp.int32),
            [pltpu.VMEM((CH, d_in), jnp.float32) for _ in range(RB)],
            pltpu.VMEM((CH, DW), jnp.float32),
            pltpu.VMEM((zrows, d_in), jnp.float32),
            pltpu.VMEM((zrows, DW), jnp.float32),
            pltpu.VMEM_SHARED((n_nodes, d_in), jnp.float32),
            pltpu.VMEM_SHARED((n_nodes, DW), jnp.float32),
            [pltpu.SemaphoreType.DMA for _ in range(RB)],
            [pltpu.SemaphoreType.DMA for _ in range(RB)],
            pltpu.SemaphoreType.DMA,
            pltpu.SemaphoreType.DMA,
        ],
        compiler_params=pltpu.CompilerParams(use_tc_tiling_on_sc=False),
    )
    def agg_kernel(x_hbm, edge_hbm, out_hbm, deg_hbm,
                   sidx, didx, rows, ones, zbuf, zbufd, acc, deg,
                   gsem, ssem, dsem, hsem):
        c = lax.axis_index("c")
        s = lax.axis_index("s")
        wid = c * NS + s
        ebase = wid * et
        nbase = s * rows_per_tile

        def sid(j):
            return sidx.at[pl.ds(j * CH, CH)]

        def did(j):
            return didx.at[pl.ds(j * CH, CH)]

        @pl.loop(0, zrows)
        def _zero_rows(r):
            for j in range(d_in // 16):
                zbuf[r, pl.ds(j * 16, 16)] = jnp.zeros((16,), jnp.float32)
            zbufd[r, :] = jnp.zeros((16,), jnp.float32)

        @pl.loop(0, CH)
        def _one_rows(r):
            ones[r, :] = jnp.ones((16,), jnp.float32)

        @pl.loop(0, rows_per_tile // zrows)
        def _zero_acc(i):
            pltpu.async_copy(zbuf, acc.at[pl.ds(nbase + i * zrows, zrows)],
                             hsem)
            pltpu.async_copy(zbufd, deg.at[pl.ds(nbase + i * zrows, zrows)],
                             hsem)

        @pl.loop(0, rows_per_tile // zrows)
        def _zero_drain(i):
            pltpu.make_async_copy(zbuf, acc.at[pl.ds(nbase, zrows)],
                                  hsem).wait()
            pltpu.make_async_copy(zbufd, deg.at[pl.ds(nbase, zrows)],
                                  hsem).wait()

        plsc.subcore_barrier()

        @pl.loop(0, nblk)
        def _block(b):
            e0 = ebase + b * blk_e
            pltpu.async_copy(edge_hbm.at[0, pl.ds(e0, blk_e)], sidx, hsem)
            pltpu.async_copy(edge_hbm.at[1, pl.ds(e0, blk_e)], didx, hsem)
            pltpu.make_async_copy(edge_hbm.at[0, pl.ds(e0, blk_e)], sidx,
                                  hsem).wait()
            pltpu.make_async_copy(edge_hbm.at[1, pl.ds(e0, blk_e)], didx,
                                  hsem).wait()

            for k in range(RB - 1):
                pltpu.async_copy(x_hbm.at[sid(k)], rows[k], gsem[k])

            @pl.loop(0, nrnd)
            def _round(t):
                j0 = RB * t
                for k in range(RB):
                    j = j0 + k
                    pltpu.make_async_copy(x_hbm.at[sid(j)], rows[k],
                                          gsem[k]).wait()
                    pltpu.async_copy(rows[k], acc.at[did(j)], ssem[k],
                                     add=True)
                    pltpu.async_copy(ones, deg.at[did(j)], dsem, add=True)
                    kp = (k + RB - 1) % RB
                    if k == 0:
                        @pl.when(t > 0)
                        def _w():
                            pltpu.make_async_copy(rows[kp], acc.at[did(0)],
                                                  ssem[kp]).wait()
                    else:
                        pltpu.make_async_copy(rows[kp], acc.at[did(0)],
                                              ssem[kp]).wait()
                    pltpu.make_async_copy(ones, deg.at[did(0)], dsem).wait()
                    if k == 0:
                        pltpu.async_copy(x_hbm.at[sid(j + RB - 1)], rows[kp],
                                         gsem[kp])
                    else:
                        @pl.when(t < nrnd - 1)
                        def _g():
                            pltpu.async_copy(x_hbm.at[sid(j + RB - 1)],
                                             rows[kp], gsem[kp])

            pltpu.make_async_copy(rows[RB - 1], acc.at[did(0)],
                                  ssem[RB - 1]).wait()

        plsc.subcore_barrier()

        @pl.loop(0, rows_per_tile // drows)
        def _dump(i):
            r0 = nbase + i * drows
            pltpu.async_copy(acc.at[pl.ds(r0, drows)],
                             out_hbm.at[c, pl.ds(r0, drows)], hsem)

        pltpu.async_copy(deg.at[pl.ds(nbase, rows_per_tile)],
                         deg_hbm.at[c, pl.ds(nbase, rows_per_tile)], hsem)

        @pl.loop(0, rows_per_tile // drows)
        def _dump_drain(i):
            pltpu.make_async_copy(acc.at[pl.ds(nbase, drows)],
                                  out_hbm.at[c, pl.ds(nbase, drows)],
                                  hsem).wait()

        pltpu.make_async_copy(deg.at[pl.ds(nbase, rows_per_tile)],
                              deg_hbm.at[c, pl.ds(nbase, rows_per_tile)],
                              hsem).wait()

    return agg_kernel


def _tc_root(n_nodes, d_in, d_out, blk):
    grid = (n_nodes // blk,)

    def body(x_ref, wr_ref, b_ref, o_ref):
        o_ref[...] = jnp.dot(
            x_ref[...], wr_ref[...],
            preferred_element_type=jnp.float32) + b_ref[...]

    return pl.pallas_call(
        body,
        grid=grid,
        in_specs=[
            pl.BlockSpec((blk, d_in), lambda i: (i, 0)),
            pl.BlockSpec((d_in, d_out), lambda i: (0, 0)),
            pl.BlockSpec((1, d_out), lambda i: (0, 0)),
        ],
        out_specs=pl.BlockSpec((blk, d_out), lambda i: (i, 0)),
        out_shape=jax.ShapeDtypeStruct((n_nodes, d_out), jnp.float32),
    )


def _tc_combine(n_nodes, d_in, d_out, blk):
    grid = (n_nodes // blk,)

    def body(root_ref, pagg_ref, pdeg_ref, wn_ref, o_ref):
        ps = pagg_ref[0] + pagg_ref[1]
        deg = pdeg_ref[0, :, :1] + pdeg_ref[1, :, :1]
        agg = ps / jnp.maximum(deg, 1.0)
        o_ref[...] = root_ref[...] + jnp.dot(
            agg, wn_ref[...], preferred_element_type=jnp.float32)

    return pl.pallas_call(
        body,
        grid=grid,
        in_specs=[
            pl.BlockSpec((blk, d_out), lambda i: (i, 0)),
            pl.BlockSpec((NC, blk, d_in), lambda i: (0, i, 0)),
            pl.BlockSpec((NC, blk, DW), lambda i: (0, i, 0)),
            pl.BlockSpec((d_in, d_out), lambda i: (0, 0)),
        ],
        out_specs=pl.BlockSpec((blk, d_out), lambda i: (i, 0)),
        out_shape=jax.ShapeDtypeStruct((n_nodes, d_out), jnp.float32),
    )


def kernel(x, edge_index, W_root, W_neigh, b):
    n, d_in = x.shape
    e = edge_index.shape[1]
    d_out = W_root.shape[1]

    pagg, pdeg = _sc_aggregate(n, d_in, e)(x, edge_index)
    root = _tc_root(n, d_in, d_out, 1000)(x, W_root, b.reshape(1, d_out))
    return _tc_combine(n, d_in, d_out, 1000)(root, pagg, pdeg, W_neigh)


# speedup vs baseline: 1.0155x; 1.0124x over previous
import functools

import jax
import jax.numpy as jnp
from jax import lax
from jax.experimental import pallas as pl
from jax.experimental.pallas import tpu as pltpu
from jax.experimental.pallas import tpu_sc as plsc

NC, NS = 2, 16
NW = NC * NS
CH = 40
NB = 50
RB = 5
DW = 16


def _sc_aggregate(n_nodes, d_in, n_edges):
    et = n_edges // NW
    nch = et // CH
    nblk = nch // NB
    blk_e = NB * CH
    nrnd = NB // RB
    rows_per_tile = n_nodes // NS
    zrows = 25
    drows = 125
    assert et % CH == 0 and nch % NB == 0 and NB % RB == 0
    assert rows_per_tile % zrows == 0 and rows_per_tile % drows == 0

    mesh = plsc.VectorSubcoreMesh(
        core_axis_name="c", subcore_axis_name="s", num_cores=NC,
        num_subcores=NS)

    @functools.partial(
        pl.kernel,
        out_type=(jax.ShapeDtypeStruct((NC, n_nodes, d_in), jnp.float32),
                  jax.ShapeDtypeStruct((NC, n_nodes, DW), jnp.float32)),
        mesh=mesh,
        scratch_types=[
            pltpu.VMEM((blk_e,), jnp.int32),
            pltpu.VMEM((blk_e,), jnp.int32),
            [pltpu.VMEM((CH, d_in), jnp.float32) for _ in range(RB)],
            pltpu.VMEM((CH, DW), jnp.float32),
            pltpu.VMEM((zrows, d_in), jnp.float32),
            pltpu.VMEM((zrows, DW), jnp.float32),
            pltpu.VMEM_SHARED((n_nodes, d_in), jnp.float32),
            pltpu.VMEM_SHARED((n_nodes, DW), jnp.float32),
            [pltpu.SemaphoreType.DMA for _ in range(RB)],
            [pltpu.SemaphoreType.DMA for _ in range(RB)],
            pltpu.SemaphoreType.DMA,
            pltpu.SemaphoreType.DMA,
        ],
        compiler_params=pltpu.CompilerParams(use_tc_tiling_on_sc=False),
    )
    def agg_kernel(x_hbm, edge_hbm, out_hbm, deg_hbm,
                   sidx, didx, rows, ones, zbuf, zbufd, acc, deg,
                   gsem, ssem, dsem, hsem):
        c = lax.axis_index("c")
        s = lax.axis_index("s")
        wid = c * NS + s
        ebase = wid * et
        nbase = s * rows_per_tile

        def sid(j):
            return sidx.at[pl.ds(j * CH, CH)]

        def did(j):
            return didx.at[pl.ds(j * CH, CH)]

        @pl.loop(0, zrows)
        def _zero_rows(r):
            for j in range(d_in // 16):
                zbuf[r, pl.ds(j * 16, 16)] = jnp.zeros((16,), jnp.float32)
            zbufd[r, :] = jnp.zeros((16,), jnp.float32)

        @pl.loop(0, CH)
        def _one_rows(r):
            ones[r, :] = jnp.ones((16,), jnp.float32)

        @pl.loop(0, rows_per_tile // zrows)
        def _zero_acc(i):
            pltpu.async_copy(zbuf, acc.at[pl.ds(nbase + i * zrows, zrows)],
                             hsem)
            pltpu.async_copy(zbufd, deg.at[pl.ds(nbase + i * zrows, zrows)],
                             hsem)

        @pl.loop(0, rows_per_tile // zrows)
        def _zero_drain(i):
            pltpu.make_async_copy(zbuf, acc.at[pl.ds(nbase, zrows)],
                                  hsem).wait()
            pltpu.make_async_copy(zbufd, deg.at[pl.ds(nbase, zrows)],
                                  hsem).wait()

        plsc.subcore_barrier()

        @pl.loop(0, nblk)
        def _block(b):
            e0 = ebase + b * blk_e
            pltpu.async_copy(edge_hbm.at[0, pl.ds(e0, blk_e)], sidx, hsem)
            pltpu.async_copy(edge_hbm.at[1, pl.ds(e0, blk_e)], didx, hsem)
            pltpu.make_async_copy(edge_hbm.at[0, pl.ds(e0, blk_e)], sidx,
                                  hsem).wait()
            pltpu.make_async_copy(edge_hbm.at[1, pl.ds(e0, blk_e)], didx,
                                  hsem).wait()

            for k in range(RB - 1):
                pltpu.async_copy(x_hbm.at[sid(k)], rows[k], gsem[k])

            @pl.loop(0, nrnd)
            def _round(t):
                j0 = RB * t
                for k in range(RB):
                    j = j0 + k
                    pltpu.make_async_copy(x_hbm.at[sid(j)], rows[k],
                                          gsem[k]).wait()
                    pltpu.async_copy(rows[k], acc.at[did(j)], ssem[k],
                                     add=True)
                    pltpu.async_copy(ones, deg.at[did(j)], dsem, add=True)
                    kp = (k + RB - 1) % RB
                    if k == 0:
                        @pl.when(t > 0)
                        def _w():
                            pltpu.make_async_copy(rows[kp], acc.at[did(0)],
                                                  ssem[kp]).wait()
                    else:
                        pltpu.make_async_copy(rows[kp], acc.at[did(0)],
                                              ssem[kp]).wait()
                    pltpu.make_async_copy(ones, deg.at[did(0)], dsem).wait()
                    if k == 0:
                        pltpu.async_copy(x_hbm.at[sid(j + RB - 1)], rows[kp],
                                         gsem[kp])
                    else:
                        @pl.when(t < nrnd - 1)
                        def _g():
                            pltpu.async_copy(x_hbm.at[sid(j + RB - 1)],
                                             rows[kp], gsem[kp])

            pltpu.make_async_copy(rows[RB - 1], acc.at[did(0)],
                                  ssem[RB - 1]).wait()

        plsc.subcore_barrier()

        @pl.loop(0, rows_per_tile // drows)
        def _dump(i):
            r0 = nbase + i * drows
            pltpu.async_copy(acc.at[pl.ds(r0, drows)],
                             out_hbm.at[c, pl.ds(r0, drows)], hsem)

        pltpu.async_copy(deg.at[pl.ds(nbase, rows_per_tile)],
                         deg_hbm.at[c, pl.ds(nbase, rows_per_tile)], hsem)

        @pl.loop(0, rows_per_tile // drows)
        def _dump_drain(i):
            pltpu.make_async_copy(acc.at[pl.ds(nbase, drows)],
                                  out_hbm.at[c, pl.ds(nbase, drows)],
                                  hsem).wait()

        pltpu.make_async_copy(deg.at[pl.ds(nbase, rows_per_tile)],
                              deg_hbm.at[c, pl.ds(nbase, rows_per_tile)],
                              hsem).wait()

    return agg_kernel


def _tc_root(n_nodes, d_in, d_out, blk):
    grid = (n_nodes // blk,)

    def body(x_ref, wr_ref, b_ref, o_ref):
        o_ref[...] = jnp.dot(
            x_ref[...], wr_ref[...],
            preferred_element_type=jnp.float32) + b_ref[...]

    return pl.pallas_call(
        body,
        grid=grid,
        in_specs=[
            pl.BlockSpec((blk, d_in), lambda i: (i, 0)),
            pl.BlockSpec((d_in, d_out), lambda i: (0, 0)),
            pl.BlockSpec((1, d_out), lambda i: (0, 0)),
        ],
        out_specs=pl.BlockSpec((blk, d_out), lambda i: (i, 0)),
        out_shape=jax.ShapeDtypeStruct((n_nodes, d_out), jnp.float32),
    )


def _tc_combine(n_nodes, d_in, d_out, blk):
    grid = (n_nodes // blk,)

    def body(root_ref, pagg_ref, pdeg_ref, wn_ref, o_ref):
        ps = pagg_ref[0] + pagg_ref[1]
        deg = pdeg_ref[0, :, :1] + pdeg_ref[1, :, :1]
        agg = ps / jnp.maximum(deg, 1.0)
        o_ref[...] = root_ref[...] + jnp.dot(
            agg, wn_ref[...], preferred_element_type=jnp.float32)

    return pl.pallas_call(
        body,
        grid=grid,
        in_specs=[
            pl.BlockSpec((blk, d_out), lambda i: (i, 0)),
            pl.BlockSpec((NC, blk, d_in), lambda i: (0, i, 0)),
            pl.BlockSpec((NC, blk, DW), lambda i: (0, i, 0)),
            pl.BlockSpec((d_in, d_out), lambda i: (0, 0)),
        ],
        out_specs=pl.BlockSpec((blk, d_out), lambda i: (i, 0)),
        out_shape=jax.ShapeDtypeStruct((n_nodes, d_out), jnp.float32),
    )


def kernel(x, edge_index, W_root, W_neigh, b):
    n, d_in = x.shape
    e = edge_index.shape[1]
    d_out = W_root.shape[1]

    pagg, pdeg = _sc_aggregate(n, d_in, e)(x, edge_index)
    root = _tc_root(n, d_in, d_out, 2000)(x, W_root, b.reshape(1, d_out))
    return _tc_combine(n, d_in, d_out, 2000)(root, pagg, pdeg, W_neigh)
